# Initial kernel scaffold; baseline (speedup 1.0000x reference)
import functools

import jax
import jax.numpy as jnp
from jax import lax
from jax.experimental import pallas as pl
from jax.experimental.pallas import tpu as pltpu
from jax.experimental.pallas import tpu_sc as plsc

B, S, D, W = 4, 8192, 768, 4096
NC, NS, L = 2, 16, 16
DH = D // 2
CHUNK = 128
TOK_PER_TILE = S // NS
N_CHUNKS = TOK_PER_TILE // CHUNK
W_PER_TILE = W // NS
WBLK = 64
N_WBLK = W_PER_TILE // WBLK
N_ROUNDS = (B * 2) // NC


def _pool_body(emb_hbm, ids_hbm, out_hbm,
               ids_v, tok_v, stage_v, zbuf_v, zcnt_v, ones_v, cnt_v,
               acc_sh, cnt_sh):
    c = lax.axis_index("c")
    s = lax.axis_index("s")

    zvec = jnp.zeros((L,), jnp.float32)
    ovec = jnp.ones((L,), jnp.float32)

    def _init_row(i, _):
        for j in range(DH // L):
            zbuf_v[i, pl.ds(j * L, L)] = zvec
        return 0
    lax.fori_loop(0, WBLK, _init_row, 0)

    def _init_small(i, _):
        zcnt_v[i, :] = zvec
        ones_v[i % CHUNK, :] = ovec
        return 0
    lax.fori_loop(0, W_PER_TILE, _init_small, 0)

    for r in range(N_ROUNDS):
        g = c * N_ROUNDS + r
        b = g // 2
        hoff = (g % 2) * DH

        w_base = s * W_PER_TILE
        for blk in range(N_WBLK):
            pltpu.sync_copy(zbuf_v, acc_sh.at[pl.ds(w_base + blk * WBLK, WBLK)])
        pltpu.sync_copy(zcnt_v, cnt_sh.at[pl.ds(w_base, W_PER_TILE)])

        plsc.subcore_barrier()

        for k in range(N_CHUNKS):
            t0 = s * TOK_PER_TILE + k * CHUNK
            pltpu.sync_copy(ids_hbm.at[b, pl.ds(t0, CHUNK)], ids_v)
            pltpu.sync_copy(emb_hbm.at[b, pl.ds(t0, CHUNK), pl.ds(hoff, DH)],
                            tok_v)
            pltpu.sync_copy(tok_v, acc_sh.at[ids_v], add=True)
            pltpu.sync_copy(ones_v, cnt_sh.at[ids_v], add=True)

        plsc.subcore_barrier()

        pltpu.sync_copy(cnt_sh.at[pl.ds(w_base, W_PER_TILE)], cnt_v)
        for blk in range(N_WBLK):
            w0 = w_base + blk * WBLK
            pltpu.sync_copy(acc_sh.at[pl.ds(w0, WBLK)], stage_v)

            def _div_row(i, _, blk=blk):
                cnt = cnt_v[blk * WBLK + i, :]
                recip = 1.0 / jnp.maximum(cnt, 1.0)
                for j in range(DH // L):
                    stage_v[i, pl.ds(j * L, L)] = (
                        stage_v[i, pl.ds(j * L, L)] * recip)
                return 0
            lax.fori_loop(0, WBLK, _div_row, 0)

            pltpu.sync_copy(stage_v, out_hbm.at[b, pl.ds(w0, WBLK),
                                                pl.ds(hoff, DH)])


@jax.jit
def _pool(token_embeds, token_to_words):
    mesh = plsc.VectorSubcoreMesh(core_axis_name="c", subcore_axis_name="s",
                                  num_cores=NC, num_subcores=NS)
    kern = functools.partial(
        pl.kernel,
        out_type=jax.ShapeDtypeStruct((B, W, D), jnp.float32),
        mesh=mesh,
        scratch_types=[
            pltpu.VMEM((CHUNK,), jnp.int32),
            pltpu.VMEM((CHUNK, DH), jnp.float32),
            pltpu.VMEM((WBLK, DH), jnp.float32),
            pltpu.VMEM((WBLK, DH), jnp.float32),
            pltpu.VMEM((W_PER_TILE, L), jnp.float32),
            pltpu.VMEM((CHUNK, L), jnp.float32),
            pltpu.VMEM((W_PER_TILE, L), jnp.float32),
            pltpu.VMEM_SHARED((W, DH), jnp.float32),
            pltpu.VMEM_SHARED((W, L), jnp.float32),
        ],
    )(_pool_body)
    return kern(token_embeds, token_to_words)


def kernel(token_embeds, token_to_words):
    return _pool(token_embeds, token_to_words)


# speedup vs baseline: 2.4944x; 2.4944x over previous
import functools

import jax
import jax.numpy as jnp
from jax import lax
from jax.experimental import pallas as pl
from jax.experimental.pallas import tpu as pltpu
from jax.experimental.pallas import tpu_sc as plsc

B, S, D, W = 4, 8192, 768, 4096
NC, NS, L = 2, 16, 16
DSL = 128
N_SLICES = D // DSL
CHUNK = 128
TOK_PER_TILE = S // NS
N_CHUNKS = TOK_PER_TILE // CHUNK
W_PER_TILE = W // NS
WBLK = 64
N_WBLK = W_PER_TILE // WBLK
B_PER_SC = B // NC
N_ROUNDS = B_PER_SC * N_SLICES


def _pool_body(emb_hbm, ids_hbm, out_hbm,
               ids_v, tok_v, stage_v, zbuf_v, ones_v, cnt_v,
               acc_sh, cnt_sh):
    c = lax.axis_index("c")
    s = lax.axis_index("s")

    zvec = jnp.zeros((L,), jnp.float32)
    ovec = jnp.ones((L,), jnp.float32)

    def _init_zrow(i, _):
        for j in range(DSL // L):
            zbuf_v[i, pl.ds(j * L, L)] = zvec
        return 0
    lax.fori_loop(0, WBLK, _init_zrow, 0)

    def _init_orow(i, _):
        for j in range(DSL // L):
            ones_v[i, pl.ds(j * L, L)] = ovec
        return 0
    lax.fori_loop(0, CHUNK, _init_orow, 0)

    w_base = s * W_PER_TILE
    for r in range(N_ROUNDS):
        b = c * B_PER_SC + r // N_SLICES
        doff = (r % N_SLICES) * DSL
        first_slice = (r % N_SLICES) == 0

        for blk in range(N_WBLK):
            pltpu.sync_copy(zbuf_v, acc_sh.at[pl.ds(w_base + blk * WBLK, WBLK)])
            if first_slice:
                pltpu.sync_copy(zbuf_v,
                                cnt_sh.at[pl.ds(w_base + blk * WBLK, WBLK)])

        plsc.subcore_barrier()

        for k in range(N_CHUNKS):
            t0 = s * TOK_PER_TILE + k * CHUNK
            pltpu.sync_copy(ids_hbm.at[b, pl.ds(t0, CHUNK)], ids_v)
            pltpu.sync_copy(emb_hbm.at[b, pl.ds(t0, CHUNK), pl.ds(doff, DSL)],
                            tok_v)
            pltpu.sync_copy(tok_v, acc_sh.at[ids_v], add=True)
            if first_slice:
                pltpu.sync_copy(ones_v, cnt_sh.at[ids_v], add=True)

        plsc.subcore_barrier()

        for blk in range(N_WBLK):
            w0 = w_base + blk * WBLK
            pltpu.sync_copy(acc_sh.at[pl.ds(w0, WBLK)], stage_v)
            pltpu.sync_copy(cnt_sh.at[pl.ds(w0, WBLK)], cnt_v)

            def _div_row(i, _):
                cnt = cnt_v[i, pl.ds(0, L)]
                recip = 1.0 / jnp.maximum(cnt, 1.0)
                for j in range(DSL // L):
                    stage_v[i, pl.ds(j * L, L)] = (
                        stage_v[i, pl.ds(j * L, L)] * recip)
                return 0
            lax.fori_loop(0, WBLK, _div_row, 0)

            pltpu.sync_copy(stage_v, out_hbm.at[b, pl.ds(w0, WBLK),
                                                pl.ds(doff, DSL)])


@jax.jit
def _pool(token_embeds, token_to_words):
    mesh = plsc.VectorSubcoreMesh(core_axis_name="c", subcore_axis_name="s",
                                  num_cores=NC, num_subcores=NS)
    kern = functools.partial(
        pl.kernel,
        out_type=jax.ShapeDtypeStruct((B, W, D), jnp.float32),
        mesh=mesh,
        scratch_types=[
            pltpu.VMEM((CHUNK,), jnp.int32),
            pltpu.VMEM((CHUNK, DSL), jnp.float32),
            pltpu.VMEM((WBLK, DSL), jnp.float32),
            pltpu.VMEM((WBLK, DSL), jnp.float32),
            pltpu.VMEM((CHUNK, DSL), jnp.float32),
            pltpu.VMEM((WBLK, DSL), jnp.float32),
            pltpu.VMEM_SHARED((W, DSL), jnp.float32),
            pltpu.VMEM_SHARED((W, DSL), jnp.float32),
        ],
    )(_pool_body)
    return kern(token_embeds, token_to_words)


def kernel(token_embeds, token_to_words):
    return _pool(token_embeds, token_to_words)


# speedup vs baseline: 3.1115x; 1.2474x over previous
import functools

import jax
import jax.numpy as jnp
from jax import lax
from jax.experimental import pallas as pl
from jax.experimental.pallas import tpu as pltpu
from jax.experimental.pallas import tpu_sc as plsc

B, S, D, W = 4, 8192, 768, 4096
NC, NS, L = 2, 16, 16
DSL = 128
N_SLICES = D // DSL
CHUNK = 64
TOK_PER_TILE = S // NS
N_CHUNKS = TOK_PER_TILE // CHUNK
W_PER_TILE = W // NS
WBLK = 64
N_WBLK = W_PER_TILE // WBLK
B_PER_SC = B // NC
N_ROUNDS = B_PER_SC * N_SLICES


def _pool_body(emb_hbm, ids_hbm, out_hbm,
               ids2_v, tok2_v, stage2_v, cnt_v, zbuf_v, ones_v,
               acc_sh, cnt_sh,
               gsem, ssem, zsem, wsem, csem):
    c = lax.axis_index("c")
    s = lax.axis_index("s")

    zvec = jnp.zeros((L,), jnp.float32)
    ovec = jnp.ones((L,), jnp.float32)

    def _init_row(i, _):
        for j in range(DSL // L):
            zbuf_v[i, pl.ds(j * L, L)] = zvec
            ones_v[i, pl.ds(j * L, L)] = ovec
        return 0
    lax.fori_loop(0, WBLK, _init_row, 0)

    w_base = s * W_PER_TILE

    zeros0 = []
    for blk in range(N_WBLK):
        zeros0.append(pltpu.async_copy(
            zbuf_v, acc_sh.at[pl.ds(w_base + blk * WBLK, WBLK)], zsem))
        zeros0.append(pltpu.async_copy(
            zbuf_v, cnt_sh.at[pl.ds(w_base + blk * WBLK, WBLK)], csem))
    for d in zeros0:
        d.wait()

    plsc.subcore_barrier()

    for r in range(N_ROUNDS):
        b = c * B_PER_SC + r // N_SLICES
        dslice = r % N_SLICES
        doff = dslice * DSL
        first_slice = dslice == 0
        last_slice = dslice == N_SLICES - 1

        ids_bufs = [ids2_v.at[0], ids2_v.at[1]]
        tok_bufs = [tok2_v.at[0], tok2_v.at[1]]
        g_pend = [[], []]
        s_pend = [[], []]

        def _fire_gather(k):
            slot = k % 2
            t0 = s * TOK_PER_TILE + k * CHUNK
            g_pend[slot].append(pltpu.async_copy(
                ids_hbm.at[b, pl.ds(t0, CHUNK)], ids_bufs[slot], gsem))
            g_pend[slot].append(pltpu.async_copy(
                emb_hbm.at[b, pl.ds(t0, CHUNK), pl.ds(doff, DSL)],
                tok_bufs[slot], gsem))

        _fire_gather(0)
        for k in range(N_CHUNKS):
            slot = k % 2
            other = 1 - slot
            for d in g_pend[slot]:
                d.wait()
            g_pend[slot] = []
            if k + 1 < N_CHUNKS:
                for d in s_pend[other]:
                    d.wait()
                s_pend[other] = []
                _fire_gather(k + 1)
            s_pend[slot].append(pltpu.async_copy(
                tok_bufs[slot], acc_sh.at[ids_bufs[slot]], ssem, add=True))
            if first_slice:
                s_pend[slot].append(pltpu.async_copy(
                    ones_v, cnt_sh.at[ids_bufs[slot]], csem, add=True))
        for slot in range(2):
            for d in s_pend[slot]:
                d.wait()

        plsc.subcore_barrier()

        stage_bufs = [stage2_v.at[0], stage2_v.at[1]]
        l_pend = [[], []]
        w_pend = [[], []]
        z_pend = []

        def _fire_stage_load(blk):
            slot = blk % 2
            w0 = w_base + blk * WBLK
            l_pend[slot].append(pltpu.async_copy(
                acc_sh.at[pl.ds(w0, WBLK)], stage_bufs[slot], gsem))

        def _fire_cnt_load(blk):
            w0 = w_base + blk * WBLK
            l_pend[blk % 2].append(pltpu.async_copy(
                cnt_sh.at[pl.ds(w0, WBLK)], cnt_v, gsem))

        _fire_stage_load(0)
        _fire_cnt_load(0)
        for blk in range(N_WBLK):
            slot = blk % 2
            w0 = w_base + blk * WBLK
            for d in l_pend[slot]:
                d.wait()
            l_pend[slot] = []
            z_pend.append(pltpu.async_copy(
                zbuf_v, acc_sh.at[pl.ds(w0, WBLK)], zsem))
            if last_slice:
                z_pend.append(pltpu.async_copy(
                    zbuf_v, cnt_sh.at[pl.ds(w0, WBLK)], csem))
            if blk + 1 < N_WBLK:
                for d in w_pend[1 - slot]:
                    d.wait()
                w_pend[1 - slot] = []
                _fire_stage_load(blk + 1)

            stage = stage_bufs[slot]

            def _div_row(i, _, stage=stage):
                cvec = cnt_v[i, pl.ds(0, L)]
                recip = 1.0 / jnp.maximum(cvec, 1.0)
                for j in range(DSL // L):
                    stage[i, pl.ds(j * L, L)] = (
                        stage[i, pl.ds(j * L, L)] * recip)
                return 0
            lax.fori_loop(0, WBLK, _div_row, 0)

            if blk + 1 < N_WBLK:
                _fire_cnt_load(blk + 1)

            w_pend[slot].append(pltpu.async_copy(
                stage, out_hbm.at[b, pl.ds(w0, WBLK), pl.ds(doff, DSL)],
                wsem))

        for slot in range(2):
            for d in w_pend[slot]:
                d.wait()
        for d in z_pend:
            d.wait()

        plsc.subcore_barrier()


@jax.jit
def _pool(token_embeds, token_to_words):
    mesh = plsc.VectorSubcoreMesh(core_axis_name="c", subcore_axis_name="s",
                                  num_cores=NC, num_subcores=NS)
    kern = functools.partial(
        pl.kernel,
        out_type=jax.ShapeDtypeStruct((B, W, D), jnp.float32),
        mesh=mesh,
        scratch_types=[
            pltpu.VMEM((2, CHUNK), jnp.int32),
            pltpu.VMEM((2, CHUNK, DSL), jnp.float32),
            pltpu.VMEM((2, WBLK, DSL), jnp.float32),
            pltpu.VMEM((WBLK, DSL), jnp.float32),
            pltpu.VMEM((WBLK, DSL), jnp.float32),
            pltpu.VMEM((WBLK, DSL), jnp.float32),
            pltpu.VMEM_SHARED((W, DSL), jnp.float32),
            pltpu.VMEM_SHARED((W, DSL), jnp.float32),
            pltpu.SemaphoreType.DMA,
            pltpu.SemaphoreType.DMA,
            pltpu.SemaphoreType.DMA,
            pltpu.SemaphoreType.DMA,
            pltpu.SemaphoreType.DMA,
        ],
    )(_pool_body)
    return kern(token_embeds, token_to_words)


def kernel(token_embeds, token_to_words):
    return _pool(token_embeds, token_to_words)


# speedup vs baseline: 3.3985x; 1.0923x over previous
import functools

import jax
import jax.numpy as jnp
from jax import lax
from jax.experimental import pallas as pl
from jax.experimental.pallas import tpu as pltpu
from jax.experimental.pallas import tpu_sc as plsc

B, S, D, W = 4, 8192, 768, 4096
NC, NS, L = 2, 16, 16
DSL = 128
N_SLICES = D // DSL
CHUNK = 64
TOK_PER_TILE = S // NS
N_CHUNKS = TOK_PER_TILE // CHUNK
W_PER_TILE = W // NS
WBLK = 64
N_WBLK = W_PER_TILE // WBLK
B_PER_SC = B // NC
N_ROUNDS = B_PER_SC * N_SLICES


def _pool_body(emb_hbm, ids_hbm, out_hbm,
               ids2_v, tok2_v, stage2_v, cw_v, zbuf_v,
               acc_sh, cnt_sh,
               gsems, ssems, lsems, wsems, clsem, zsem, csem):
    c = lax.axis_index("c")
    s = lax.axis_index("s")

    zvec = jnp.zeros((L,), jnp.float32)
    ovec = jnp.ones((L,), jnp.float32)

    def _init_row(i, _):
        for j in range(DSL // L):
            zbuf_v[i, pl.ds(j * L, L)] = zvec
        return 0
    lax.fori_loop(0, WBLK, _init_row, 0)

    w_base = s * W_PER_TILE

    zeros0 = []
    for blk in range(N_WBLK):
        zeros0.append(pltpu.async_copy(
            zbuf_v, acc_sh.at[pl.ds(w_base + blk * WBLK, WBLK)], zsem))
        zeros0.append(pltpu.async_copy(
            zbuf_v, cnt_sh.at[pl.ds(w_base + blk * WBLK, WBLK)], csem))
    for d in zeros0:
        d.wait()

    ids_bufs = [ids2_v.at[0], ids2_v.at[1]]
    tok_bufs = [tok2_v.at[0], tok2_v.at[1]]
    g_pend = [[], []]

    def _round_params(r):
        return c * B_PER_SC + r // N_SLICES, (r % N_SLICES) * DSL

    def _fire_gather(r, k):
        b_, doff_ = _round_params(r)
        slot = k % 2
        t0 = s * TOK_PER_TILE + k * CHUNK
        g_pend[slot].append(pltpu.async_copy(
            ids_hbm.at[b_, pl.ds(t0, CHUNK)], ids_bufs[slot], gsems[slot]))
        g_pend[slot].append(pltpu.async_copy(
            emb_hbm.at[b_, pl.ds(t0, CHUNK), pl.ds(doff_, DSL)],
            tok_bufs[slot], gsems[slot]))

    for k in range(2):
        _fire_gather(0, k)

    plsc.subcore_barrier()

    for r in range(N_ROUNDS):
        b, doff = _round_params(r)
        dslice = r % N_SLICES
        first_slice = dslice == 0
        last_slice = dslice == N_SLICES - 1

        if first_slice:
            def _init_ones(i, _):
                for j in range(DSL // L):
                    cw_v[i, pl.ds(j * L, L)] = ovec
                return 0
            lax.fori_loop(0, CHUNK, _init_ones, 0)

        s_pend = [[], []]
        c_pend = []
        for k in range(N_CHUNKS):
            slot = k % 2
            other = 1 - slot
            for d in g_pend[slot]:
                d.wait()
            g_pend[slot] = []
            if 2 <= k + 1 < N_CHUNKS:
                for d in s_pend[other]:
                    d.wait()
                s_pend[other] = []
                _fire_gather(r, k + 1)
            s_pend[slot].append(pltpu.async_copy(
                tok_bufs[slot], acc_sh.at[ids_bufs[slot]], ssems[slot],
                add=True))
            if first_slice:
                c_pend.append(pltpu.async_copy(
                    cw_v, cnt_sh.at[ids_bufs[slot]], csem, add=True))
        for slot in range(2):
            for d in s_pend[slot]:
                d.wait()
        for d in c_pend:
            d.wait()

        if r + 1 < N_ROUNDS:
            for k in range(2):
                _fire_gather(r + 1, k)

        plsc.subcore_barrier()

        stage_bufs = [stage2_v.at[0], stage2_v.at[1]]
        l_pend = [[], []]
        w_pend = [[], []]
        z_pend = []

        def _fire_stage_load(blk):
            slot = blk % 2
            w0 = w_base + blk * WBLK
            l_pend[slot].append(pltpu.async_copy(
                acc_sh.at[pl.ds(w0, WBLK)], stage_bufs[slot], lsems[slot]))

        def _fire_cnt_load(blk):
            w0 = w_base + blk * WBLK
            l_pend[blk % 2].append(pltpu.async_copy(
                cnt_sh.at[pl.ds(w0, WBLK)], cw_v, clsem))

        _fire_stage_load(0)
        _fire_cnt_load(0)
        for blk in range(N_WBLK):
            slot = blk % 2
            w0 = w_base + blk * WBLK
            for d in l_pend[slot]:
                d.wait()
            l_pend[slot] = []
            z_pend.append(pltpu.async_copy(
                zbuf_v, acc_sh.at[pl.ds(w0, WBLK)], zsem))
            if last_slice:
                z_pend.append(pltpu.async_copy(
                    zbuf_v, cnt_sh.at[pl.ds(w0, WBLK)], csem))
            if blk + 1 < N_WBLK:
                for d in w_pend[1 - slot]:
                    d.wait()
                w_pend[1 - slot] = []
                _fire_stage_load(blk + 1)

            stage = stage_bufs[slot]

            def _div_row(i, _, stage=stage):
                cvec = cw_v[i, pl.ds(0, L)]
                recip = 1.0 / jnp.maximum(cvec, 1.0)
                for j in range(DSL // L):
                    stage[i, pl.ds(j * L, L)] = (
                        stage[i, pl.ds(j * L, L)] * recip)
                return 0
            lax.fori_loop(0, WBLK, _div_row, 0)

            if blk + 1 < N_WBLK:
                _fire_cnt_load(blk + 1)

            w_pend[slot].append(pltpu.async_copy(
                stage, out_hbm.at[b, pl.ds(w0, WBLK), pl.ds(doff, DSL)],
                wsems[slot]))

        for slot in range(2):
            for d in w_pend[slot]:
                d.wait()
        for d in z_pend:
            d.wait()

        plsc.subcore_barrier()


@jax.jit
def _pool(token_embeds, token_to_words):
    mesh = plsc.VectorSubcoreMesh(core_axis_name="c", subcore_axis_name="s",
                                  num_cores=NC, num_subcores=NS)
    kern = functools.partial(
        pl.kernel,
        out_type=jax.ShapeDtypeStruct((B, W, D), jnp.float32),
        mesh=mesh,
        scratch_types=[
            pltpu.VMEM((2, CHUNK), jnp.int32),
            pltpu.VMEM((2, CHUNK, DSL), jnp.float32),
            pltpu.VMEM((2, WBLK, DSL), jnp.float32),
            pltpu.VMEM((CHUNK, DSL), jnp.float32),
            pltpu.VMEM((WBLK, DSL), jnp.float32),
            pltpu.VMEM_SHARED((W, DSL), jnp.float32),
            pltpu.VMEM_SHARED((W, DSL), jnp.float32),
            [pltpu.SemaphoreType.DMA] * 2,
            [pltpu.SemaphoreType.DMA] * 2,
            [pltpu.SemaphoreType.DMA] * 2,
            [pltpu.SemaphoreType.DMA] * 2,
            pltpu.SemaphoreType.DMA,
            pltpu.SemaphoreType.DMA,
            pltpu.SemaphoreType.DMA,
        ],
    )(_pool_body)
    return kern(token_embeds, token_to_words)


def kernel(token_embeds, token_to_words):
    return _pool(token_embeds, token_to_words)


# speedup vs baseline: 3.9640x; 1.1664x over previous
import functools

import jax
import jax.numpy as jnp
from jax import lax
from jax.experimental import pallas as pl
from jax.experimental.pallas import tpu as pltpu
from jax.experimental.pallas import tpu_sc as plsc

B, S, D, W = 4, 8192, 768, 4096
NC, NS, L = 2, 16, 16
DSL = 128
N_SLICES = D // DSL
CHUNK = 64
TOK_PER_TILE = S // NS
N_CHUNKS = TOK_PER_TILE // CHUNK
W_PER_TILE = W // NS
WBLK = 32
N_WBLK = W_PER_TILE // WBLK
B_PER_SC = B // NC
N_ROUNDS = B_PER_SC * N_SLICES


NSLOT = 4


def _pool_body(emb_hbm, ids_hbm, out_hbm,
               ids2_v, tok2_v, stage2_v, cw_v, zbuf_v,
               acc_sh, cnt_sh,
               gsems, ssems, lsems, wsems, clsem, zsem, csem):
    c = lax.axis_index("c")
    s = lax.axis_index("s")

    zvec = jnp.zeros((L,), jnp.float32)
    ovec = jnp.ones((L,), jnp.float32)

    def _init_row(i, _):
        for j in range(DSL // L):
            zbuf_v[i, pl.ds(j * L, L)] = zvec
        return 0
    lax.fori_loop(0, WBLK, _init_row, 0)

    w_base = s * W_PER_TILE

    zeros0 = []
    for blk in range(N_WBLK):
        zeros0.append(pltpu.async_copy(
            zbuf_v, acc_sh.at[pl.ds(w_base + blk * WBLK, WBLK)], zsem))
        zeros0.append(pltpu.async_copy(
            zbuf_v, cnt_sh.at[pl.ds(w_base + blk * WBLK, WBLK)], csem))
    for d in zeros0:
        d.wait()

    ids_bufs = [ids2_v.at[i] for i in range(NSLOT)]
    tok_bufs = [tok2_v.at[i] for i in range(NSLOT)]
    g_pend = [[] for _ in range(NSLOT)]

    def _round_params(r):
        return c * B_PER_SC + r // N_SLICES, (r % N_SLICES) * DSL

    def _fire_gather(r, k):
        b_, doff_ = _round_params(r)
        slot = k % NSLOT
        t0 = s * TOK_PER_TILE + k * CHUNK
        g_pend[slot].append(pltpu.async_copy(
            ids_hbm.at[b_, pl.ds(t0, CHUNK)], ids_bufs[slot], gsems[slot]))
        g_pend[slot].append(pltpu.async_copy(
            emb_hbm.at[b_, pl.ds(t0, CHUNK), pl.ds(doff_, DSL)],
            tok_bufs[slot], gsems[slot]))

    for k in range(NSLOT):
        _fire_gather(0, k)

    plsc.subcore_barrier()

    for r in range(N_ROUNDS):
        b, doff = _round_params(r)
        dslice = r % N_SLICES
        first_slice = dslice == 0
        last_slice = dslice == N_SLICES - 1

        if first_slice:
            def _init_ones(i, _):
                for j in range(DSL // L):
                    cw_v[i, pl.ds(j * L, L)] = ovec
                return 0
            lax.fori_loop(0, CHUNK, _init_ones, 0)

        s_pend = [[] for _ in range(NSLOT)]
        c_pend = []
        for k in range(N_CHUNKS):
            slot = k % NSLOT
            for d in g_pend[slot]:
                d.wait()
            g_pend[slot] = []
            s_pend[slot].append(pltpu.async_copy(
                tok_bufs[slot], acc_sh.at[ids_bufs[slot]], ssems[slot],
                add=True))
            if first_slice:
                c_pend.append(pltpu.async_copy(
                    cw_v, cnt_sh.at[ids_bufs[slot]], csem, add=True))
            nxt = k + NSLOT
            if nxt < N_CHUNKS:
                nslot = nxt % NSLOT
                for d in s_pend[nslot]:
                    d.wait()
                s_pend[nslot] = []
                _fire_gather(r, nxt)
        for slot in range(NSLOT):
            for d in s_pend[slot]:
                d.wait()
        for d in c_pend:
            d.wait()

        if r + 1 < N_ROUNDS:
            for k in range(NSLOT):
                _fire_gather(r + 1, k)

        plsc.subcore_barrier()

        stage_bufs = [stage2_v.at[0], stage2_v.at[1]]
        l_pend = [[], []]
        w_pend = [[], []]
        z_pend = []

        def _fire_stage_load(blk):
            slot = blk % 2
            w0 = w_base + blk * WBLK
            l_pend[slot].append(pltpu.async_copy(
                acc_sh.at[pl.ds(w0, WBLK)], stage_bufs[slot], lsems[slot]))

        def _fire_cnt_load(blk):
            w0 = w_base + blk * WBLK
            l_pend[blk % 2].append(pltpu.async_copy(
                cnt_sh.at[pl.ds(w0, WBLK)], cw_v.at[pl.ds(0, WBLK)], clsem))

        _fire_stage_load(0)
        _fire_cnt_load(0)
        for blk in range(N_WBLK):
            slot = blk % 2
            w0 = w_base + blk * WBLK
            for d in l_pend[slot]:
                d.wait()
            l_pend[slot] = []
            z_pend.append(pltpu.async_copy(
                zbuf_v, acc_sh.at[pl.ds(w0, WBLK)], zsem))
            if last_slice:
                z_pend.append(pltpu.async_copy(
                    zbuf_v, cnt_sh.at[pl.ds(w0, WBLK)], csem))
            if blk + 1 < N_WBLK:
                for d in w_pend[1 - slot]:
                    d.wait()
                w_pend[1 - slot] = []
                _fire_stage_load(blk + 1)

            stage = stage_bufs[slot]

            def _div_row(i, _, stage=stage):
                cvec = cw_v[i, pl.ds(0, L)]
                recip = 1.0 / jnp.maximum(cvec, 1.0)
                for j in range(DSL // L):
                    stage[i, pl.ds(j * L, L)] = (
                        stage[i, pl.ds(j * L, L)] * recip)
                return 0
            lax.fori_loop(0, WBLK, _div_row, 0)

            if blk + 1 < N_WBLK:
                _fire_cnt_load(blk + 1)

            w_pend[slot].append(pltpu.async_copy(
                stage, out_hbm.at[b, pl.ds(w0, WBLK), pl.ds(doff, DSL)],
                wsems[slot]))

        for slot in range(2):
            for d in w_pend[slot]:
                d.wait()
        for d in z_pend:
            d.wait()

        plsc.subcore_barrier()


@jax.jit
def _pool(token_embeds, token_to_words):
    mesh = plsc.VectorSubcoreMesh(core_axis_name="c", subcore_axis_name="s",
                                  num_cores=NC, num_subcores=NS)
    kern = functools.partial(
        pl.kernel,
        out_type=jax.ShapeDtypeStruct((B, W, D), jnp.float32),
        mesh=mesh,
        scratch_types=[
            pltpu.VMEM((4, CHUNK), jnp.int32),
            pltpu.VMEM((4, CHUNK, DSL), jnp.float32),
            pltpu.VMEM((2, WBLK, DSL), jnp.float32),
            pltpu.VMEM((CHUNK, DSL), jnp.float32),
            pltpu.VMEM((WBLK, DSL), jnp.float32),
            pltpu.VMEM_SHARED((W, DSL), jnp.float32),
            pltpu.VMEM_SHARED((W, DSL), jnp.float32),
            [pltpu.SemaphoreType.DMA] * 4,
            [pltpu.SemaphoreType.DMA] * 4,
            [pltpu.SemaphoreType.DMA] * 2,
            [pltpu.SemaphoreType.DMA] * 2,
            pltpu.SemaphoreType.DMA,
            pltpu.SemaphoreType.DMA,
            pltpu.SemaphoreType.DMA,
        ],
    )(_pool_body)
    return kern(token_embeds, token_to_words)


def kernel(token_embeds, token_to_words):
    return _pool(token_embeds, token_to_words)


# speedup vs baseline: 3.9922x; 1.0071x over previous
import functools

import jax
import jax.numpy as jnp
from jax import lax
from jax.experimental import pallas as pl
from jax.experimental.pallas import tpu as pltpu
from jax.experimental.pallas import tpu_sc as plsc

B, S, D, W = 4, 8192, 768, 4096
NC, NS, L = 2, 16, 16
DSL = 128
N_SLICES = D // DSL
CHUNK = 64
TOK_PER_TILE = S // NS
N_CHUNKS = TOK_PER_TILE // CHUNK
W_PER_TILE = W // NS
WBLK = 32
N_WBLK = W_PER_TILE // WBLK
B_PER_SC = B // NC
N_ROUNDS = B_PER_SC * N_SLICES


NSLOT = 5


def _pool_body(emb_hbm, ids_hbm, out_hbm,
               ids2_v, tok2_v, stage2_v, cw_v, zbuf_v,
               acc_sh, cnt_sh,
               gsems, ssems, lsems, wsems, clsem, zsem, csem):
    c = lax.axis_index("c")
    s = lax.axis_index("s")

    zvec = jnp.zeros((L,), jnp.float32)
    ovec = jnp.ones((L,), jnp.float32)

    def _init_row(i, _):
        for j in range(DSL // L):
            zbuf_v[i, pl.ds(j * L, L)] = zvec
        return 0
    lax.fori_loop(0, WBLK, _init_row, 0)

    w_base = s * W_PER_TILE

    zeros0 = []
    for blk in range(N_WBLK):
        zeros0.append(pltpu.async_copy(
            zbuf_v, acc_sh.at[pl.ds(w_base + blk * WBLK, WBLK)], zsem))
        zeros0.append(pltpu.async_copy(
            zbuf_v, cnt_sh.at[pl.ds(w_base + blk * WBLK, WBLK)], csem))
    for d in zeros0:
        d.wait()

    ids_bufs = [ids2_v.at[i] for i in range(NSLOT)]
    tok_bufs = [tok2_v.at[i] for i in range(NSLOT)]
    g_pend = [[] for _ in range(NSLOT)]

    def _round_params(r):
        return c * B_PER_SC + r // N_SLICES, (r % N_SLICES) * DSL

    def _fire_gather(r, k):
        b_, doff_ = _round_params(r)
        slot = k % NSLOT
        t0 = s * TOK_PER_TILE + k * CHUNK
        g_pend[slot].append(pltpu.async_copy(
            ids_hbm.at[b_, pl.ds(t0, CHUNK)], ids_bufs[slot], gsems[slot]))
        g_pend[slot].append(pltpu.async_copy(
            emb_hbm.at[b_, pl.ds(t0, CHUNK), pl.ds(doff_, DSL)],
            tok_bufs[slot], gsems[slot]))

    for k in range(NSLOT):
        _fire_gather(0, k)

    plsc.subcore_barrier()

    for r in range(N_ROUNDS):
        b, doff = _round_params(r)
        dslice = r % N_SLICES
        first_slice = dslice == 0
        last_slice = dslice == N_SLICES - 1

        if first_slice:
            def _init_ones(i, _):
                for j in range(DSL // L):
                    cw_v[i, pl.ds(j * L, L)] = ovec
                return 0
            lax.fori_loop(0, CHUNK, _init_ones, 0)

        s_pend = [[] for _ in range(NSLOT)]
        c_pend = []
        for k in range(N_CHUNKS):
            slot = k % NSLOT
            for d in g_pend[slot]:
                d.wait()
            g_pend[slot] = []
            s_pend[slot].append(pltpu.async_copy(
                tok_bufs[slot], acc_sh.at[ids_bufs[slot]], ssems[slot],
                add=True))
            if first_slice:
                c_pend.append(pltpu.async_copy(
                    cw_v, cnt_sh.at[ids_bufs[slot]], csem, add=True))
            nxt = k + NSLOT
            if nxt < N_CHUNKS:
                nslot = nxt % NSLOT
                for d in s_pend[nslot]:
                    d.wait()
                s_pend[nslot] = []
                _fire_gather(r, nxt)
        for slot in range(NSLOT):
            for d in s_pend[slot]:
                d.wait()
        for d in c_pend:
            d.wait()

        if r + 1 < N_ROUNDS:
            for k in range(NSLOT):
                _fire_gather(r + 1, k)

        plsc.subcore_barrier()

        stage_bufs = [stage2_v.at[0], stage2_v.at[1]]
        l_pend = [[], []]
        w_pend = [[], []]
        z_pend = []

        def _fire_stage_load(blk):
            slot = blk % 2
            w0 = w_base + blk * WBLK
            l_pend[slot].append(pltpu.async_copy(
                acc_sh.at[pl.ds(w0, WBLK)], stage_bufs[slot], lsems[slot]))

        def _fire_cnt_load(blk):
            w0 = w_base + blk * WBLK
            l_pend[blk % 2].append(pltpu.async_copy(
                cnt_sh.at[pl.ds(w0, WBLK)], cw_v.at[pl.ds(0, WBLK)], clsem))

        _fire_stage_load(0)
        _fire_cnt_load(0)
        for blk in range(N_WBLK):
            slot = blk % 2
            w0 = w_base + blk * WBLK
            for d in l_pend[slot]:
                d.wait()
            l_pend[slot] = []
            z_pend.append(pltpu.async_copy(
                zbuf_v, acc_sh.at[pl.ds(w0, WBLK)], zsem))
            if last_slice:
                z_pend.append(pltpu.async_copy(
                    zbuf_v, cnt_sh.at[pl.ds(w0, WBLK)], csem))
            if blk + 1 < N_WBLK:
                for d in w_pend[1 - slot]:
                    d.wait()
                w_pend[1 - slot] = []
                _fire_stage_load(blk + 1)

            stage = stage_bufs[slot]

            def _div_row(i, _, stage=stage):
                cvec = cw_v[i, pl.ds(0, L)]
                recip = 1.0 / jnp.maximum(cvec, 1.0)
                for j in range(DSL // L):
                    stage[i, pl.ds(j * L, L)] = (
                        stage[i, pl.ds(j * L, L)] * recip)
                return 0
            lax.fori_loop(0, WBLK, _div_row, 0)

            if blk + 1 < N_WBLK:
                _fire_cnt_load(blk + 1)

            w_pend[slot].append(pltpu.async_copy(
                stage, out_hbm.at[b, pl.ds(w0, WBLK), pl.ds(doff, DSL)],
                wsems[slot]))

        for slot in range(2):
            for d in w_pend[slot]:
                d.wait()
        for d in z_pend:
            d.wait()

        plsc.subcore_barrier()


@jax.jit
def _pool(token_embeds, token_to_words):
    mesh = plsc.VectorSubcoreMesh(core_axis_name="c", subcore_axis_name="s",
                                  num_cores=NC, num_subcores=NS)
    kern = functools.partial(
        pl.kernel,
        out_type=jax.ShapeDtypeStruct((B, W, D), jnp.float32),
        mesh=mesh,
        scratch_types=[
            pltpu.VMEM((5, CHUNK), jnp.int32),
            pltpu.VMEM((5, CHUNK, DSL), jnp.float32),
            pltpu.VMEM((2, WBLK, DSL), jnp.float32),
            pltpu.VMEM((CHUNK, DSL), jnp.float32),
            pltpu.VMEM((WBLK, DSL), jnp.float32),
            pltpu.VMEM_SHARED((W, DSL), jnp.float32),
            pltpu.VMEM_SHARED((W, DSL), jnp.float32),
            [pltpu.SemaphoreType.DMA] * 5,
            [pltpu.SemaphoreType.DMA] * 5,
            [pltpu.SemaphoreType.DMA] * 2,
            [pltpu.SemaphoreType.DMA] * 2,
            pltpu.SemaphoreType.DMA,
            pltpu.SemaphoreType.DMA,
            pltpu.SemaphoreType.DMA,
        ],
    )(_pool_body)
    return kern(token_embeds, token_to_words)


def kernel(token_embeds, token_to_words):
    return _pool(token_embeds, token_to_words)


# speedup vs baseline: 4.2294x; 1.0594x over previous
import functools

import jax
import jax.numpy as jnp
from jax import lax
from jax.experimental import pallas as pl
from jax.experimental.pallas import tpu as pltpu
from jax.experimental.pallas import tpu_sc as plsc

B, S, D, W = 4, 8192, 768, 4096
NC, NS, L = 2, 16, 16
DSL = 128
N_SLICES = D // DSL
CHUNK = 64
TOK_PER_TILE = S // NS
N_CHUNKS = TOK_PER_TILE // CHUNK
W_PER_TILE = W // NS
WBLK = 32
N_WBLK = W_PER_TILE // WBLK
B_PER_SC = B // NC
N_ROUNDS = B_PER_SC * N_SLICES


NSLOT = 4


def _pool_body(emb_hbm, ids_hbm, out_hbm,
               ids2_v, tok2_v, stage4_v, cw_v, zbuf_v,
               acc_sh, cnt_sh,
               gsems, ssems, lsems, wsems, clsems, zsem, csem):
    c = lax.axis_index("c")
    s = lax.axis_index("s")

    zvec = jnp.zeros((L,), jnp.float32)
    ovec = jnp.ones((L,), jnp.float32)

    def _init_row(i, _):
        for j in range(DSL // L):
            zbuf_v[i, pl.ds(j * L, L)] = zvec
        return 0
    lax.fori_loop(0, WBLK, _init_row, 0)

    w_base = s * W_PER_TILE

    zeros0 = []
    for blk in range(N_WBLK):
        zeros0.append(pltpu.async_copy(
            zbuf_v, acc_sh.at[pl.ds(w_base + blk * WBLK, WBLK)], zsem))
        zeros0.append(pltpu.async_copy(
            zbuf_v, cnt_sh.at[pl.ds(w_base + blk * WBLK, WBLK)], csem))
    for d in zeros0:
        d.wait()

    ids_bufs = [ids2_v.at[i] for i in range(NSLOT)]
    tok_bufs = [tok2_v.at[i] for i in range(NSLOT)]
    g_pend = [[] for _ in range(NSLOT)]

    def _round_params(r):
        return c * B_PER_SC + r // N_SLICES, (r % N_SLICES) * DSL

    def _fire_gather(r, k):
        b_, doff_ = _round_params(r)
        slot = k % NSLOT
        t0 = s * TOK_PER_TILE + k * CHUNK
        g_pend[slot].append(pltpu.async_copy(
            ids_hbm.at[b_, pl.ds(t0, CHUNK)], ids_bufs[slot], gsems[slot]))
        g_pend[slot].append(pltpu.async_copy(
            emb_hbm.at[b_, pl.ds(t0, CHUNK), pl.ds(doff_, DSL)],
            tok_bufs[slot], gsems[slot]))

    for k in range(NSLOT):
        _fire_gather(0, k)

    plsc.subcore_barrier()

    for r in range(N_ROUNDS):
        b, doff = _round_params(r)
        dslice = r % N_SLICES
        first_slice = dslice == 0
        last_slice = dslice == N_SLICES - 1

        if first_slice:
            def _init_ones(i, _):
                for j in range(DSL // L):
                    cw_v[i, pl.ds(j * L, L)] = ovec
                return 0
            lax.fori_loop(0, CHUNK, _init_ones, 0)

        s_pend = [[] for _ in range(NSLOT)]
        c_pend = []
        for k in range(N_CHUNKS):
            slot = k % NSLOT
            for d in g_pend[slot]:
                d.wait()
            g_pend[slot] = []
            s_pend[slot].append(pltpu.async_copy(
                tok_bufs[slot], acc_sh.at[ids_bufs[slot]], ssems[slot],
                add=True))
            if first_slice:
                c_pend.append(pltpu.async_copy(
                    cw_v, cnt_sh.at[ids_bufs[slot]], csem, add=True))
            nxt = k + NSLOT
            if nxt < N_CHUNKS:
                nslot = nxt % NSLOT
                for d in s_pend[nslot]:
                    d.wait()
                s_pend[nslot] = []
                _fire_gather(r, nxt)
        for slot in range(NSLOT):
            for d in s_pend[slot]:
                d.wait()
        for d in c_pend:
            d.wait()

        if r + 1 < N_ROUNDS:
            for k in range(NSLOT):
                _fire_gather(r + 1, k)

        plsc.subcore_barrier()

        SSLOT = 4
        stage_bufs = [stage4_v.at[i] for i in range(SSLOT)]
        cnt_bufs = [cw_v.at[pl.ds(0, WBLK)], cw_v.at[pl.ds(WBLK, WBLK)]]
        l_pend = [[] for _ in range(SSLOT)]
        c_pendl = [[], []]
        w_pend = [[] for _ in range(SSLOT)]
        z_pend = []

        def _fire_stage_load(blk):
            slot = blk % SSLOT
            w0 = w_base + blk * WBLK
            l_pend[slot].append(pltpu.async_copy(
                acc_sh.at[pl.ds(w0, WBLK)], stage_bufs[slot], lsems[slot]))

        def _fire_cnt_load(blk):
            w0 = w_base + blk * WBLK
            c_pendl[blk % 2].append(pltpu.async_copy(
                cnt_sh.at[pl.ds(w0, WBLK)], cnt_bufs[blk % 2],
                clsems[blk % 2]))

        for blk in range(min(SSLOT, N_WBLK)):
            _fire_stage_load(blk)
        for blk in range(min(2, N_WBLK)):
            _fire_cnt_load(blk)
        for blk in range(N_WBLK):
            slot = blk % SSLOT
            w0 = w_base + blk * WBLK
            for d in l_pend[slot]:
                d.wait()
            l_pend[slot] = []
            for d in c_pendl[blk % 2]:
                d.wait()
            c_pendl[blk % 2] = []
            z_pend.append(pltpu.async_copy(
                zbuf_v, acc_sh.at[pl.ds(w0, WBLK)], zsem))
            if last_slice:
                z_pend.append(pltpu.async_copy(
                    zbuf_v, cnt_sh.at[pl.ds(w0, WBLK)], csem))
            nxt = blk + SSLOT
            if nxt < N_WBLK:
                for d in w_pend[nxt % SSLOT]:
                    d.wait()
                w_pend[nxt % SSLOT] = []
                _fire_stage_load(nxt)

            stage = stage_bufs[slot]
            cbase = (blk % 2) * WBLK

            def _div_row(i, _, stage=stage, cbase=cbase):
                cvec = cw_v[cbase + i, pl.ds(0, L)]
                recip = 1.0 / jnp.maximum(cvec, 1.0)
                for j in range(DSL // L):
                    stage[i, pl.ds(j * L, L)] = (
                        stage[i, pl.ds(j * L, L)] * recip)
                return 0
            lax.fori_loop(0, WBLK, _div_row, 0)

            if blk + 2 < N_WBLK:
                _fire_cnt_load(blk + 2)

            w_pend[slot].append(pltpu.async_copy(
                stage, out_hbm.at[b, pl.ds(w0, WBLK), pl.ds(doff, DSL)],
                wsems[slot]))

        for slot in range(SSLOT):
            for d in w_pend[slot]:
                d.wait()
        for d in z_pend:
            d.wait()

        plsc.subcore_barrier()


@jax.jit
def _pool(token_embeds, token_to_words):
    mesh = plsc.VectorSubcoreMesh(core_axis_name="c", subcore_axis_name="s",
                                  num_cores=NC, num_subcores=NS)
    kern = functools.partial(
        pl.kernel,
        out_type=jax.ShapeDtypeStruct((B, W, D), jnp.float32),
        mesh=mesh,
        scratch_types=[
            pltpu.VMEM((4, CHUNK), jnp.int32),
            pltpu.VMEM((4, CHUNK, DSL), jnp.float32),
            pltpu.VMEM((4, WBLK, DSL), jnp.float32),
            pltpu.VMEM((CHUNK, DSL), jnp.float32),
            pltpu.VMEM((WBLK, DSL), jnp.float32),
            pltpu.VMEM_SHARED((W, DSL), jnp.float32),
            pltpu.VMEM_SHARED((W, DSL), jnp.float32),
            [pltpu.SemaphoreType.DMA] * 4,
            [pltpu.SemaphoreType.DMA] * 4,
            [pltpu.SemaphoreType.DMA] * 4,
            [pltpu.SemaphoreType.DMA] * 4,
            [pltpu.SemaphoreType.DMA] * 2,
            pltpu.SemaphoreType.DMA,
            pltpu.SemaphoreType.DMA,
        ],
    )(_pool_body)
    return kern(token_embeds, token_to_words)


def kernel(token_embeds, token_to_words):
    return _pool(token_embeds, token_to_words)


# speedup vs baseline: 4.2487x; 1.0045x over previous
import functools

import jax
import jax.numpy as jnp
from jax import lax
from jax.experimental import pallas as pl
from jax.experimental.pallas import tpu as pltpu
from jax.experimental.pallas import tpu_sc as plsc

B, S, D, W = 4, 8192, 768, 4096
NC, NS, L = 2, 16, 16
DSL = 128
N_SLICES = D // DSL
CHUNK = 64
TOK_PER_TILE = S // NS
N_CHUNKS = TOK_PER_TILE // CHUNK
W_PER_TILE = W // NS
WBLK = 32
N_WBLK = W_PER_TILE // WBLK
B_PER_SC = B // NC
N_ROUNDS = B_PER_SC * N_SLICES


NSLOT = 4


def _pool_body(emb_hbm, ids_hbm, out_hbm,
               ids2_v, tok2_v, stage4_v, cw_v, zbuf_v,
               acc_sh, cnt_sh,
               gsems, ssems, lsems, wsems, clsems, zsem, csem):
    c = lax.axis_index("c")
    s = lax.axis_index("s")

    zvec = jnp.zeros((L,), jnp.float32)
    ovec = jnp.ones((L,), jnp.float32)

    def _init_row(i, _):
        for j in range(DSL // L):
            zbuf_v[i, pl.ds(j * L, L)] = zvec
        return 0
    lax.fori_loop(0, WBLK, _init_row, 0)

    w_base = s * W_PER_TILE

    zeros0 = []
    for blk in range(N_WBLK):
        zeros0.append(pltpu.async_copy(
            zbuf_v, acc_sh.at[pl.ds(w_base + blk * WBLK, WBLK)], zsem))
        zeros0.append(pltpu.async_copy(
            zbuf_v, cnt_sh.at[pl.ds(w_base + blk * WBLK, WBLK)], csem))
    for d in zeros0:
        d.wait()

    ids_bufs = [ids2_v.at[i] for i in range(NSLOT)]
    tok_bufs = [tok2_v.at[i] for i in range(NSLOT)]
    g_pend = [[] for _ in range(NSLOT)]

    def _round_params(r):
        return c * B_PER_SC + r // N_SLICES, (r % N_SLICES) * DSL

    def _fire_gather(r, k):
        b_, doff_ = _round_params(r)
        slot = k % NSLOT
        t0 = s * TOK_PER_TILE + k * CHUNK
        g_pend[slot].append(pltpu.async_copy(
            ids_hbm.at[b_, pl.ds(t0, CHUNK)], ids_bufs[slot], gsems[slot]))
        g_pend[slot].append(pltpu.async_copy(
            emb_hbm.at[b_, pl.ds(t0, CHUNK), pl.ds(doff_, DSL)],
            tok_bufs[slot], gsems[slot]))

    for k in range(NSLOT):
        _fire_gather(0, k)

    plsc.subcore_barrier()

    for r in range(N_ROUNDS):
        b, doff = _round_params(r)
        dslice = r % N_SLICES
        first_slice = dslice == 0
        last_slice = dslice == N_SLICES - 1

        if first_slice:
            def _init_ones(i, _):
                for j in range(DSL // L):
                    cw_v[i, pl.ds(j * L, L)] = ovec
                return 0
            lax.fori_loop(0, CHUNK, _init_ones, 0)

        s_pend = [[] for _ in range(NSLOT)]
        c_pend = []
        for k in range(N_CHUNKS):
            slot = k % NSLOT
            for d in g_pend[slot]:
                d.wait()
            g_pend[slot] = []
            s_pend[slot].append(pltpu.async_copy(
                tok_bufs[slot], acc_sh.at[ids_bufs[slot]], ssems[slot],
                add=True))
            if first_slice:
                c_pend.append(pltpu.async_copy(
                    cw_v, cnt_sh.at[ids_bufs[slot]], csem, add=True))
            nxt = k + NSLOT
            if nxt < N_CHUNKS:
                nslot = nxt % NSLOT
                for d in s_pend[nslot]:
                    d.wait()
                s_pend[nslot] = []
                _fire_gather(r, nxt)
        for slot in range(NSLOT):
            for d in s_pend[slot]:
                d.wait()
        for d in c_pend:
            d.wait()

        if r + 1 < N_ROUNDS:
            for k in range(NSLOT):
                _fire_gather(r + 1, k)

        plsc.subcore_barrier()

        SSLOT = 4
        stage_bufs = [stage4_v.at[i] for i in range(SSLOT)]
        cnt_bufs = [cw_v.at[pl.ds(0, WBLK)], cw_v.at[pl.ds(WBLK, WBLK)]]
        l_pend = [[] for _ in range(SSLOT)]
        c_pendl = [[], []]
        w_pend = [[] for _ in range(SSLOT)]
        z_pend = []

        def _fire_stage_load(blk):
            slot = blk % SSLOT
            w0 = w_base + blk * WBLK
            l_pend[slot].append(pltpu.async_copy(
                acc_sh.at[pl.ds(w0, WBLK)], stage_bufs[slot], lsems[slot]))

        def _fire_cnt_load(blk):
            w0 = w_base + blk * WBLK
            c_pendl[blk % 2].append(pltpu.async_copy(
                cnt_sh.at[pl.ds(w0, WBLK)], cnt_bufs[blk % 2],
                clsems[blk % 2]))

        for blk in range(min(SSLOT - 1, N_WBLK)):
            _fire_stage_load(blk)
        for blk in range(min(2, N_WBLK)):
            _fire_cnt_load(blk)
        for blk in range(N_WBLK):
            slot = blk % SSLOT
            w0 = w_base + blk * WBLK
            for d in l_pend[slot]:
                d.wait()
            l_pend[slot] = []
            for d in c_pendl[blk % 2]:
                d.wait()
            c_pendl[blk % 2] = []
            z_pend.append(pltpu.async_copy(
                zbuf_v, acc_sh.at[pl.ds(w0, WBLK)], zsem))
            if last_slice:
                z_pend.append(pltpu.async_copy(
                    zbuf_v, cnt_sh.at[pl.ds(w0, WBLK)], csem))
            nxt = blk + SSLOT - 1
            if nxt < N_WBLK:
                for d in w_pend[nxt % SSLOT]:
                    d.wait()
                w_pend[nxt % SSLOT] = []
                _fire_stage_load(nxt)

            stage = stage_bufs[slot]
            cbase = (blk % 2) * WBLK

            def _div_row(i, _, stage=stage, cbase=cbase):
                cvec = cw_v[cbase + i, pl.ds(0, L)]
                recip = 1.0 / jnp.maximum(cvec, 1.0)
                for j in range(DSL // L):
                    stage[i, pl.ds(j * L, L)] = (
                        stage[i, pl.ds(j * L, L)] * recip)
                return 0
            lax.fori_loop(0, WBLK, _div_row, 0)

            if blk + 2 < N_WBLK:
                _fire_cnt_load(blk + 2)

            w_pend[slot].append(pltpu.async_copy(
                stage, out_hbm.at[b, pl.ds(w0, WBLK), pl.ds(doff, DSL)],
                wsems[slot]))

        for slot in range(SSLOT):
            for d in w_pend[slot]:
                d.wait()
        for d in z_pend:
            d.wait()

        plsc.subcore_barrier()


@jax.jit
def _pool(token_embeds, token_to_words):
    mesh = plsc.VectorSubcoreMesh(core_axis_name="c", subcore_axis_name="s",
                                  num_cores=NC, num_subcores=NS)
    kern = functools.partial(
        pl.kernel,
        out_type=jax.ShapeDtypeStruct((B, W, D), jnp.float32),
        mesh=mesh,
        scratch_types=[
            pltpu.VMEM((4, CHUNK), jnp.int32),
            pltpu.VMEM((4, CHUNK, DSL), jnp.float32),
            pltpu.VMEM((4, WBLK, DSL), jnp.float32),
            pltpu.VMEM((CHUNK, DSL), jnp.float32),
            pltpu.VMEM((WBLK, DSL), jnp.float32),
            pltpu.VMEM_SHARED((W, DSL), jnp.float32),
            pltpu.VMEM_SHARED((W, DSL), jnp.float32),
            [pltpu.SemaphoreType.DMA] * 4,
            [pltpu.SemaphoreType.DMA] * 4,
            [pltpu.SemaphoreType.DMA] * 4,
            [pltpu.SemaphoreType.DMA] * 4,
            [pltpu.SemaphoreType.DMA] * 2,
            pltpu.SemaphoreType.DMA,
            pltpu.SemaphoreType.DMA,
        ],
    )(_pool_body)
    return kern(token_embeds, token_to_words)


def kernel(token_embeds, token_to_words):
    return _pool(token_embeds, token_to_words)


# speedup vs baseline: 4.2593x; 1.0025x over previous
import functools

import jax
import jax.numpy as jnp
from jax import lax
from jax.experimental import pallas as pl
from jax.experimental.pallas import tpu as pltpu
from jax.experimental.pallas import tpu_sc as plsc

B, S, D, W = 4, 8192, 768, 4096
NC, NS, L = 2, 16, 16
DSL = 128
N_SLICES = D // DSL
CHUNK = 64
TOK_PER_TILE = S // NS
N_CHUNKS = TOK_PER_TILE // CHUNK
W_PER_TILE = W // NS
WBLK = 32
N_WBLK = W_PER_TILE // WBLK
B_PER_SC = B // NC
N_ROUNDS = B_PER_SC * N_SLICES


NSLOT = 4


def _pool_body(emb_hbm, ids_hbm, out_hbm,
               ids_all_v, tok2_v, stage4_v, cw_v, zbuf_v,
               acc_sh, cnt_sh,
               gsems, ssems, lsems, wsems, clsems, zsem, csem):
    c = lax.axis_index("c")
    s = lax.axis_index("s")

    zvec = jnp.zeros((L,), jnp.float32)
    ovec = jnp.ones((L,), jnp.float32)

    def _init_row(i, _):
        for j in range(DSL // L):
            zbuf_v[i, pl.ds(j * L, L)] = zvec
        return 0
    lax.fori_loop(0, WBLK, _init_row, 0)

    w_base = s * W_PER_TILE

    zeros0 = []
    for blk in range(N_WBLK):
        zeros0.append(pltpu.async_copy(
            zbuf_v, acc_sh.at[pl.ds(w_base + blk * WBLK, WBLK)], zsem))
        zeros0.append(pltpu.async_copy(
            zbuf_v, cnt_sh.at[pl.ds(w_base + blk * WBLK, WBLK)], csem))
    for d in zeros0:
        d.wait()

    ids_rows = [ids_all_v.at[k] for k in range(N_CHUNKS)]
    tok_bufs = [tok2_v.at[i] for i in range(NSLOT)]
    g_pend = [[] for _ in range(NSLOT)]

    def _round_params(r):
        return c * B_PER_SC + r // N_SLICES, (r % N_SLICES) * DSL

    def _fire_gather(r, k):
        b_, doff_ = _round_params(r)
        slot = k % NSLOT
        t0 = s * TOK_PER_TILE + k * CHUNK
        g_pend[slot].append(pltpu.async_copy(
            emb_hbm.at[b_, pl.ds(t0, CHUNK), pl.ds(doff_, DSL)],
            tok_bufs[slot], gsems[slot]))

    for k in range(NSLOT):
        _fire_gather(0, k)

    plsc.subcore_barrier()

    for r in range(N_ROUNDS):
        b, doff = _round_params(r)
        dslice = r % N_SLICES
        first_slice = dslice == 0
        last_slice = dslice == N_SLICES - 1

        if first_slice:
            i_pend = []
            for k in range(N_CHUNKS):
                t0 = s * TOK_PER_TILE + k * CHUNK
                i_pend.append(pltpu.async_copy(
                    ids_hbm.at[b, pl.ds(t0, CHUNK)], ids_rows[k], csem))
            for d in i_pend:
                d.wait()
            def _init_ones(i, _):
                for j in range(DSL // L):
                    cw_v[i, pl.ds(j * L, L)] = ovec
                return 0
            lax.fori_loop(0, CHUNK, _init_ones, 0)

        s_pend = [[] for _ in range(NSLOT)]
        c_pend = []
        for k in range(N_CHUNKS):
            slot = k % NSLOT
            for d in g_pend[slot]:
                d.wait()
            g_pend[slot] = []
            s_pend[slot].append(pltpu.async_copy(
                tok_bufs[slot], acc_sh.at[ids_rows[k]], ssems[slot],
                add=True))
            if first_slice:
                c_pend.append(pltpu.async_copy(
                    cw_v, cnt_sh.at[ids_rows[k]], csem, add=True))
            nxt = k + NSLOT
            if nxt < N_CHUNKS:
                nslot = nxt % NSLOT
                for d in s_pend[nslot]:
                    d.wait()
                s_pend[nslot] = []
                _fire_gather(r, nxt)
        for slot in range(NSLOT):
            for d in s_pend[slot]:
                d.wait()
        for d in c_pend:
            d.wait()

        if r + 1 < N_ROUNDS:
            for k in range(NSLOT):
                _fire_gather(r + 1, k)

        plsc.subcore_barrier()

        SSLOT = 4
        stage_bufs = [stage4_v.at[i] for i in range(SSLOT)]
        cnt_bufs = [cw_v.at[pl.ds(0, WBLK)], cw_v.at[pl.ds(WBLK, WBLK)]]
        l_pend = [[] for _ in range(SSLOT)]
        c_pendl = [[], []]
        w_pend = [[] for _ in range(SSLOT)]
        z_pend = []

        def _fire_stage_load(blk):
            slot = blk % SSLOT
            w0 = w_base + blk * WBLK
            l_pend[slot].append(pltpu.async_copy(
                acc_sh.at[pl.ds(w0, WBLK)], stage_bufs[slot], lsems[slot]))

        def _fire_cnt_load(blk):
            w0 = w_base + blk * WBLK
            c_pendl[blk % 2].append(pltpu.async_copy(
                cnt_sh.at[pl.ds(w0, WBLK)], cnt_bufs[blk % 2],
                clsems[blk % 2]))

        for blk in range(min(SSLOT - 1, N_WBLK)):
            _fire_stage_load(blk)
        for blk in range(min(2, N_WBLK)):
            _fire_cnt_load(blk)
        for blk in range(N_WBLK):
            slot = blk % SSLOT
            w0 = w_base + blk * WBLK
            for d in l_pend[slot]:
                d.wait()
            l_pend[slot] = []
            for d in c_pendl[blk % 2]:
                d.wait()
            c_pendl[blk % 2] = []
            z_pend.append(pltpu.async_copy(
                zbuf_v, acc_sh.at[pl.ds(w0, WBLK)], zsem))
            if last_slice:
                z_pend.append(pltpu.async_copy(
                    zbuf_v, cnt_sh.at[pl.ds(w0, WBLK)], csem))
            nxt = blk + SSLOT - 1
            if nxt < N_WBLK:
                for d in w_pend[nxt % SSLOT]:
                    d.wait()
                w_pend[nxt % SSLOT] = []
                _fire_stage_load(nxt)

            stage = stage_bufs[slot]
            cbase = (blk % 2) * WBLK

            def _div_row(i, _, stage=stage, cbase=cbase):
                cvec = cw_v[cbase + i, pl.ds(0, L)]
                recip = 1.0 / jnp.maximum(cvec, 1.0)
                for j in range(DSL // L):
                    stage[i, pl.ds(j * L, L)] = (
                        stage[i, pl.ds(j * L, L)] * recip)
                return 0
            lax.fori_loop(0, WBLK, _div_row, 0)

            if blk + 2 < N_WBLK:
                _fire_cnt_load(blk + 2)

            w_pend[slot].append(pltpu.async_copy(
                stage, out_hbm.at[b, pl.ds(w0, WBLK), pl.ds(doff, DSL)],
                wsems[slot]))

        for slot in range(SSLOT):
            for d in w_pend[slot]:
                d.wait()
        for d in z_pend:
            d.wait()

        plsc.subcore_barrier()


@jax.jit
def _pool(token_embeds, token_to_words):
    mesh = plsc.VectorSubcoreMesh(core_axis_name="c", subcore_axis_name="s",
                                  num_cores=NC, num_subcores=NS)
    kern = functools.partial(
        pl.kernel,
        out_type=jax.ShapeDtypeStruct((B, W, D), jnp.float32),
        mesh=mesh,
        scratch_types=[
            pltpu.VMEM((N_CHUNKS, CHUNK), jnp.int32),
            pltpu.VMEM((4, CHUNK, DSL), jnp.float32),
            pltpu.VMEM((4, WBLK, DSL), jnp.float32),
            pltpu.VMEM((CHUNK, DSL), jnp.float32),
            pltpu.VMEM((WBLK, DSL), jnp.float32),
            pltpu.VMEM_SHARED((W, DSL), jnp.float32),
            pltpu.VMEM_SHARED((W, DSL), jnp.float32),
            [pltpu.SemaphoreType.DMA] * 4,
            [pltpu.SemaphoreType.DMA] * 4,
            [pltpu.SemaphoreType.DMA] * 4,
            [pltpu.SemaphoreType.DMA] * 4,
            [pltpu.SemaphoreType.DMA] * 2,
            pltpu.SemaphoreType.DMA,
            pltpu.SemaphoreType.DMA,
        ],
    )(_pool_body)
    return kern(token_embeds, token_to_words)


def kernel(token_embeds, token_to_words):
    return _pool(token_embeds, token_to_words)
